# baseline (speedup 1.0000x reference)
import functools

import jax
import jax.numpy as jnp
from jax import lax
from jax.experimental import pallas as pl
from jax.experimental.pallas import tpu as pltpu
from jax.experimental.pallas import tpu_sc as plsc


def _sc_gather(W, b, ids):
    n, hid = ids.shape[0], W.shape[1]
    info = plsc.get_sparse_core_info()
    nw = info.num_cores * info.num_subcores
    per = n // nw
    assert per * nw == n and per % 8 == 0
    mesh = plsc.VectorSubcoreMesh(core_axis_name="c", subcore_axis_name="s")

    @functools.partial(
        pl.kernel,
        out_type=(
            jax.ShapeDtypeStruct((n, hid), jnp.float32),
            jax.ShapeDtypeStruct((n,), jnp.float32),
        ),
        mesh=mesh,
        scratch_types=[
            pltpu.VMEM((per,), jnp.int32),
            pltpu.VMEM((per, hid), jnp.float32),
            pltpu.VMEM((per,), jnp.float32),
            pltpu.SemaphoreType.DMA,
            pltpu.SemaphoreType.DMA,
        ],
    )
    def gather_kernel(w_hbm, b_hbm, ids_hbm, rows_out, bias_out,
                      idx_v, rows_v, bias_v, sem_r, sem_b):
        wid = lax.axis_index("s") * info.num_cores + lax.axis_index("c")
        base = wid * per
        pltpu.sync_copy(ids_hbm.at[pl.ds(base, per)], idx_v)
        cp_r = pltpu.async_copy(w_hbm.at[idx_v], rows_v, sem_r)
        cp_b = pltpu.async_copy(b_hbm.at[idx_v], bias_v, sem_b)
        cp_r.wait()
        cp_b.wait()
        pltpu.sync_copy(rows_v, rows_out.at[pl.ds(base, per)])
        pltpu.sync_copy(bias_v, bias_out.at[pl.ds(base, per)])

    return gather_kernel(W, b, ids)


def _tc_logits(output, targets2, rows, sample_ids2, sample_b2, sample_f2,
               true_b2, true_f2, bt):
    b, hid = output.shape
    ns = sample_ids2.shape[1]

    def body(x_ref, tgt_ref, sw_ref, tw_ref, sid_ref, sb_ref, sf_ref,
             tb_ref, tf_ref, o_ref):
        x = x_ref[...]
        sw = sw_ref[...]
        logits = lax.dot_general(
            x, sw, (((1,), (1,)), ((), ())),
            preferred_element_type=jnp.float32)
        logits = logits + (sb_ref[...] - jnp.log(sf_ref[...]))
        acc = tgt_ref[...] == sid_ref[...]
        logits = jnp.where(acc, jnp.float32(-1e37), logits)
        tl = (jnp.sum(x * tw_ref[...], axis=1, keepdims=True)
              + tb_ref[...] - jnp.log(tf_ref[...]))
        o_ref[:, :ns] = logits
        o_ref[:, ns:] = tl

    grid = (b // bt,)
    return pl.pallas_call(
        body,
        grid=grid,
        in_specs=[
            pl.BlockSpec((bt, hid), lambda i: (i, 0)),
            pl.BlockSpec((bt, 1), lambda i: (i, 0)),
            pl.BlockSpec((ns, hid), lambda i: (0, 0)),
            pl.BlockSpec((bt, hid), lambda i: (ns // bt + i, 0)),
            pl.BlockSpec((1, ns), lambda i: (0, 0)),
            pl.BlockSpec((1, ns), lambda i: (0, 0)),
            pl.BlockSpec((1, ns), lambda i: (0, 0)),
            pl.BlockSpec((bt, 1), lambda i: (i, 0)),
            pl.BlockSpec((bt, 1), lambda i: (i, 0)),
        ],
        out_specs=pl.BlockSpec((bt, 1 + ns), lambda i: (i, 0)),
        out_shape=jax.ShapeDtypeStruct((b, 1 + ns), jnp.float32),
    )(output, targets2, rows, rows, sample_ids2, sample_b2, sample_f2,
      true_b2, true_f2)


def kernel(output, targets, W, b, sample_ids, true_freq, sample_freq):
    bsz, hid = output.shape
    ns = sample_ids.shape[0]
    ids = jnp.concatenate([sample_ids, targets])
    rows, bias = _sc_gather(W, b, ids)
    logits = _tc_logits(
        output,
        targets.reshape(bsz, 1),
        rows,
        sample_ids.reshape(1, ns),
        bias[:ns].reshape(1, ns),
        sample_freq.reshape(1, ns),
        bias[ns:].reshape(bsz, 1),
        true_freq.reshape(bsz, 1),
        bt=512,
    )
    new_targets = jnp.zeros((bsz,), dtype=jnp.int32)
    return logits, new_targets


# speedup vs baseline: 1.8737x; 1.8737x over previous
import functools

import jax
import jax.numpy as jnp
from jax import lax
from jax.experimental import pallas as pl
from jax.experimental.pallas import tpu as pltpu
from jax.experimental.pallas import tpu_sc as plsc


def _sc_gather(W, b, ids):
    n, hid = ids.shape[0], W.shape[1]
    info = plsc.get_sparse_core_info()
    nw = info.num_cores * info.num_subcores
    per = n // nw
    assert per * nw == n and per % 8 == 0
    mesh = plsc.VectorSubcoreMesh(core_axis_name="c", subcore_axis_name="s")

    @functools.partial(
        pl.kernel,
        out_type=(
            jax.ShapeDtypeStruct((n, hid), jnp.float32),
            jax.ShapeDtypeStruct((n,), jnp.float32),
        ),
        mesh=mesh,
        scratch_types=[
            pltpu.VMEM((per,), jnp.int32),
            pltpu.VMEM((per, hid), jnp.float32),
            pltpu.VMEM((per,), jnp.float32),
            pltpu.SemaphoreType.DMA,
            pltpu.SemaphoreType.DMA,
        ],
    )
    def gather_kernel(w_hbm, b_hbm, ids_hbm, rows_out, bias_out,
                      idx_v, rows_v, bias_v, sem_r, sem_b):
        wid = lax.axis_index("s") * info.num_cores + lax.axis_index("c")
        base = wid * per
        pltpu.sync_copy(ids_hbm.at[pl.ds(base, per)], idx_v)
        cp_r = pltpu.async_copy(w_hbm.at[idx_v], rows_v, sem_r)
        cp_b = pltpu.async_copy(b_hbm.at[idx_v], bias_v, sem_b)
        cp_r.wait()
        cp_b.wait()
        pltpu.sync_copy(rows_v, rows_out.at[pl.ds(base, per)])
        pltpu.sync_copy(bias_v, bias_out.at[pl.ds(base, per)])

    return gather_kernel(W, b, ids)


def _tc_logits_t(output, targets2, rows, sample_ids2, sample_b2, sample_f2,
                 true_b2, true_f2, bt):
    b, hid = output.shape
    ns = sample_ids2.shape[0]

    def body(x_ref, tgt_ref, sw_ref, tw_ref, sid_ref, sb_ref, sf_ref,
             tb_ref, tf_ref, o_ref):
        x = x_ref[...]
        sw = sw_ref[...]
        logits_t = lax.dot_general(
            sw, x, (((1,), (1,)), ((), ())),
            preferred_element_type=jnp.float32)
        logits_t = logits_t + (sb_ref[...] - jnp.log(sf_ref[...]))
        acc = sid_ref[...] == tgt_ref[...]
        logits_t = jnp.where(acc, jnp.float32(-1e37), logits_t)
        ones = jnp.ones((1, hid), dtype=jnp.float32)
        tl = lax.dot_general(
            ones, x * tw_ref[...], (((1,), (1,)), ((), ())),
            preferred_element_type=jnp.float32)
        tl = tl + tb_ref[...] - jnp.log(tf_ref[...])
        o_ref[...] = jnp.concatenate([tl, logits_t], axis=0)

    grid = (b // bt,)
    return pl.pallas_call(
        body,
        grid=grid,
        in_specs=[
            pl.BlockSpec((bt, hid), lambda j: (j, 0)),
            pl.BlockSpec((1, bt), lambda j: (0, j)),
            pl.BlockSpec((ns, hid), lambda j: (0, 0)),
            pl.BlockSpec((bt, hid), lambda j: (ns // bt + j, 0)),
            pl.BlockSpec((ns, 1), lambda j: (0, 0)),
            pl.BlockSpec((ns, 1), lambda j: (0, 0)),
            pl.BlockSpec((ns, 1), lambda j: (0, 0)),
            pl.BlockSpec((1, bt), lambda j: (0, j)),
            pl.BlockSpec((1, bt), lambda j: (0, j)),
        ],
        out_specs=pl.BlockSpec((1 + ns, bt), lambda j: (0, j)),
        out_shape=jax.ShapeDtypeStruct((1 + ns, b), jnp.float32),
    )(output, targets2, rows, rows, sample_ids2, sample_b2, sample_f2,
      true_b2, true_f2)


def kernel(output, targets, W, b, sample_ids, true_freq, sample_freq):
    bsz, hid = output.shape
    ns = sample_ids.shape[0]
    ids = jnp.concatenate([sample_ids, targets])
    rows, bias = _sc_gather(W, b, ids)
    logits_t = _tc_logits_t(
        output,
        targets.reshape(1, bsz),
        rows,
        sample_ids.reshape(ns, 1),
        bias[:ns].reshape(ns, 1),
        sample_freq.reshape(ns, 1),
        bias[ns:].reshape(1, bsz),
        true_freq.reshape(1, bsz),
        bt=256,
    )
    logits = logits_t.T
    new_targets = jnp.zeros((bsz,), dtype=jnp.int32)
    return logits, new_targets


# speedup vs baseline: 1.9604x; 1.0463x over previous
import functools

import jax
import jax.numpy as jnp
from jax import lax
from jax.experimental import pallas as pl
from jax.experimental.pallas import tpu as pltpu
from jax.experimental.pallas import tpu_sc as plsc


def _sc_gather(W, b, ids):
    n, hid = ids.shape[0], W.shape[1]
    info = plsc.get_sparse_core_info()
    nw = info.num_cores * info.num_subcores
    per = n // nw
    assert per * nw == n and per % 8 == 0
    mesh = plsc.VectorSubcoreMesh(core_axis_name="c", subcore_axis_name="s")

    @functools.partial(
        pl.kernel,
        out_type=(
            jax.ShapeDtypeStruct((n, hid), jnp.float32),
            jax.ShapeDtypeStruct((n,), jnp.float32),
        ),
        mesh=mesh,
        scratch_types=[
            pltpu.VMEM((per,), jnp.int32),
            pltpu.VMEM((per, hid), jnp.float32),
            pltpu.VMEM((per,), jnp.float32),
            pltpu.SemaphoreType.DMA,
            pltpu.SemaphoreType.DMA,
        ],
    )
    def gather_kernel(w_hbm, b_hbm, ids_hbm, rows_out, bias_out,
                      idx_v, rows_v, bias_v, sem_r, sem_b):
        wid = lax.axis_index("s") * info.num_cores + lax.axis_index("c")
        base = wid * per
        pltpu.sync_copy(ids_hbm.at[pl.ds(base, per)], idx_v)
        cp_r = pltpu.async_copy(w_hbm.at[idx_v], rows_v, sem_r)
        cp_b = pltpu.async_copy(b_hbm.at[idx_v], bias_v, sem_b)
        cp_r.wait()
        cp_b.wait()
        pltpu.sync_copy(rows_v, rows_out.at[pl.ds(base, per)])
        pltpu.sync_copy(bias_v, bias_out.at[pl.ds(base, per)])

    return gather_kernel(W, b, ids)


def _tc_logits_t(output, targets2, rows, class_vecs, true_b2, true_f2, bt):
    b, hid = output.shape
    ns = class_vecs.shape[0]

    def body(x_ref, tgt_ref, sw_ref, tw_ref, cv_ref, tb_ref, tf_ref, o_ref):
        x = x_ref[...]
        sw = sw_ref[...]
        logits_t = lax.dot_general(
            sw.astype(jnp.bfloat16), x.astype(jnp.bfloat16),
            (((1,), (1,)), ((), ())),
            preferred_element_type=jnp.float32)
        sid = lax.bitcast_convert_type(cv_ref[:, 0:1], jnp.int32)
        sb = cv_ref[:, 1:2]
        sf = cv_ref[:, 2:3]
        logits_t = logits_t + (sb - jnp.log(sf))
        acc = sid == tgt_ref[...]
        logits_t = jnp.where(acc, jnp.float32(-1e37), logits_t)
        ones = jnp.ones((1, hid), dtype=jnp.float32)
        tl = lax.dot_general(
            ones, x * tw_ref[...], (((1,), (1,)), ((), ())),
            preferred_element_type=jnp.float32)
        tl = tl + tb_ref[...] - jnp.log(tf_ref[...])
        o_ref[...] = jnp.concatenate([tl, logits_t], axis=0)

    grid = (b // bt,)
    return pl.pallas_call(
        body,
        grid=grid,
        in_specs=[
            pl.BlockSpec((bt, hid), lambda j: (j, 0)),
            pl.BlockSpec((1, bt), lambda j: (0, j)),
            pl.BlockSpec((ns, hid), lambda j: (0, 0)),
            pl.BlockSpec((bt, hid), lambda j: (ns // bt + j, 0)),
            pl.BlockSpec((ns, 3), lambda j: (0, 0)),
            pl.BlockSpec((1, bt), lambda j: (0, j)),
            pl.BlockSpec((1, bt), lambda j: (0, j)),
        ],
        out_specs=pl.BlockSpec((1 + ns, bt), lambda j: (0, j)),
        out_shape=jax.ShapeDtypeStruct((1 + ns, b), jnp.float32),
    )(output, targets2, rows, rows, class_vecs, true_b2, true_f2)


def kernel(output, targets, W, b, sample_ids, true_freq, sample_freq):
    bsz, hid = output.shape
    ns = sample_ids.shape[0]
    ids = jnp.concatenate([sample_ids, targets])
    rows, bias = _sc_gather(W, b, ids)
    class_vecs = jnp.stack(
        [lax.bitcast_convert_type(sample_ids, jnp.float32),
         bias[:ns], sample_freq], axis=1)
    logits_t = _tc_logits_t(
        output,
        targets.reshape(1, bsz),
        rows,
        class_vecs,
        bias[ns:].reshape(1, bsz),
        true_freq.reshape(1, bsz),
        bt=512,
    )
    logits = logits_t.T
    new_targets = jnp.zeros((bsz,), dtype=jnp.int32)
    return logits, new_targets


# speedup vs baseline: 1.9618x; 1.0007x over previous
import functools

import jax
import jax.numpy as jnp
from jax import lax
from jax.experimental import pallas as pl
from jax.experimental.pallas import tpu as pltpu
from jax.experimental.pallas import tpu_sc as plsc


def _sc_gather(W, b, ids):
    n, hid = ids.shape[0], W.shape[1]
    info = plsc.get_sparse_core_info()
    nw = info.num_cores * info.num_subcores
    per = n // nw
    assert per * nw == n and per % 8 == 0
    mesh = plsc.VectorSubcoreMesh(core_axis_name="c", subcore_axis_name="s")

    @functools.partial(
        pl.kernel,
        out_type=(
            jax.ShapeDtypeStruct((n, hid), jnp.float32),
            jax.ShapeDtypeStruct((n,), jnp.float32),
        ),
        mesh=mesh,
        scratch_types=[
            pltpu.VMEM((per,), jnp.int32),
            pltpu.VMEM((per, hid), jnp.float32),
            pltpu.VMEM((per,), jnp.float32),
            pltpu.SemaphoreType.DMA,
            pltpu.SemaphoreType.DMA,
        ],
    )
    def gather_kernel(w_hbm, b_hbm, ids_hbm, rows_out, bias_out,
                      idx_v, rows_v, bias_v, sem_r, sem_b):
        wid = lax.axis_index("s") * info.num_cores + lax.axis_index("c")
        base = wid * per
        pltpu.sync_copy(ids_hbm.at[pl.ds(base, per)], idx_v)
        cp_r = pltpu.async_copy(w_hbm.at[idx_v], rows_v, sem_r)
        cp_b = pltpu.async_copy(b_hbm.at[idx_v], bias_v, sem_b)
        cp_r.wait()
        cp_b.wait()
        pltpu.sync_copy(rows_v, rows_out.at[pl.ds(base, per)])
        pltpu.sync_copy(bias_v, bias_out.at[pl.ds(base, per)])

    return gather_kernel(W, b, ids)


def _tc_logits_t(output, targets2, rows, class_vecs, true_b2, true_f2, bt):
    b, hid = output.shape
    ns = class_vecs.shape[0]

    def body(x_ref, tgt_ref, sw_ref, tw_ref, cv_ref, tb_ref, tf_ref, o_ref):
        x = x_ref[...]
        sw = sw_ref[...]
        logits_t = lax.dot_general(
            sw.astype(jnp.bfloat16), x.astype(jnp.bfloat16),
            (((1,), (1,)), ((), ())),
            preferred_element_type=jnp.float32)
        sid = cv_ref[:, 0:1]
        sb = lax.bitcast_convert_type(cv_ref[:, 1:2], jnp.float32)
        sf = lax.bitcast_convert_type(cv_ref[:, 2:3], jnp.float32)
        logits_t = logits_t + (sb - jnp.log(sf))
        acc = sid == tgt_ref[...]
        logits_t = jnp.where(acc, jnp.float32(-1e37), logits_t)
        ones = jnp.ones((1, hid), dtype=jnp.float32)
        tl = lax.dot_general(
            ones, x * tw_ref[...], (((1,), (1,)), ((), ())),
            preferred_element_type=jnp.float32)
        tl = tl + tb_ref[...] - jnp.log(tf_ref[...])
        o_ref[...] = jnp.concatenate([tl, logits_t], axis=0)

    grid = (b // bt,)
    return pl.pallas_call(
        body,
        grid=grid,
        in_specs=[
            pl.BlockSpec((bt, hid), lambda j: (j, 0)),
            pl.BlockSpec((1, bt), lambda j: (0, j)),
            pl.BlockSpec((ns, hid), lambda j: (0, 0)),
            pl.BlockSpec((bt, hid), lambda j: (ns // bt + j, 0)),
            pl.BlockSpec((ns, 3), lambda j: (0, 0)),
            pl.BlockSpec((1, bt), lambda j: (0, j)),
            pl.BlockSpec((1, bt), lambda j: (0, j)),
        ],
        out_specs=pl.BlockSpec((1 + ns, bt), lambda j: (0, j)),
        out_shape=jax.ShapeDtypeStruct((1 + ns, b), jnp.float32),
    )(output, targets2, rows, rows, class_vecs, true_b2, true_f2)


def kernel(output, targets, W, b, sample_ids, true_freq, sample_freq):
    bsz, hid = output.shape
    ns = sample_ids.shape[0]
    ids = jnp.concatenate([sample_ids, targets])
    rows, bias = _sc_gather(W, b, ids)
    class_vecs = jnp.stack(
        [sample_ids,
         lax.bitcast_convert_type(bias[:ns], jnp.int32),
         lax.bitcast_convert_type(sample_freq, jnp.int32)], axis=1)
    logits_t = _tc_logits_t(
        output,
        targets.reshape(1, bsz),
        rows,
        class_vecs,
        bias[ns:].reshape(1, bsz),
        true_freq.reshape(1, bsz),
        bt=512,
    )
    logits = logits_t.T
    new_targets = jnp.zeros((bsz,), dtype=jnp.int32)
    return logits, new_targets
